# Initial kernel scaffold; baseline (speedup 1.0000x reference)
import jax
import jax.numpy as jnp
from jax.experimental import pallas as pl


def kernel(x, value_embedding, mark_absent, idx_offset):
    raise NotImplementedError("write your pallas kernel here")


# speedup vs baseline: 2.0070x; 2.0070x over previous
import functools

import jax
import jax.numpy as jnp
from jax import lax
from jax.experimental import pallas as pl
from jax.experimental.pallas import tpu as pltpu
from jax.experimental.pallas import tpu_sc as plsc

N_PROPERTIES = 26
N_VALUES = 40000
DIM_EMB = 64
ROWS = 4096 * 20
NC, NS, LANES = 2, 16, 16
NW = NC * NS
CHUNK = 128
ROWS_PER_W = ROWS // NW
CHUNKS_PER_W = ROWS_PER_W // CHUNK
N_CHUNKS = ROWS // CHUNK
VPR = DIM_EMB // LANES


def _sc_body(xprep_hbm, table_hbm, out_hbm, xchunk, gbuf0, gbuf1, acc,
             sem0, sem1):
  wid = lax.axis_index("s") * NC + lax.axis_index("c")
  gbufs = (gbuf0, gbuf1)
  sems = (sem0, sem1)

  def accum(buf, first):
    def body(i, _):
      for u in range(4):
        r = i * 4 + u
        for l in range(VPR):
          v = buf[r, pl.ds(16 * l, 16)]
          if first:
            acc[r, pl.ds(16 * l, 16)] = v
          else:
            plsc.addupdate(acc.at[r, pl.ds(16 * l, 16)], v)
      return 0
    lax.fori_loop(0, CHUNK // 4, body, 0)

  def chunk_body(t, _):
    c = wid * CHUNKS_PER_W + t
    pltpu.sync_copy(xprep_hbm.at[c], xchunk)

    def start_gather(j):
      off = jnp.int32(j * N_VALUES)
      for k in range(CHUNK // 16):
        sl = pl.ds(16 * k, 16)
        xchunk[j, sl] = xchunk[j, sl] + off
      return pltpu.async_copy(table_hbm.at[xchunk.at[j]], gbufs[j % 2],
                              sems[j % 2])

    cp = start_gather(0)
    for j in range(1, N_PROPERTIES):
      cp_next = start_gather(j)
      cp.wait()
      accum(gbufs[(j - 1) % 2], first=(j == 1))
      cp = cp_next
    cp.wait()
    accum(gbufs[(N_PROPERTIES - 1) % 2], first=False)

    pltpu.sync_copy(acc, out_hbm.at[pl.ds(c * CHUNK, CHUNK)])
    return 0

  lax.fori_loop(0, CHUNKS_PER_W, chunk_body, 0)


def _sc_gather_pool(xprep, table):
  mesh = plsc.VectorSubcoreMesh(core_axis_name="c", subcore_axis_name="s")
  return pl.kernel(
      _sc_body,
      out_type=jax.ShapeDtypeStruct((ROWS, DIM_EMB), jnp.float32),
      mesh=mesh,
      scratch_types=[
          pltpu.VMEM((N_PROPERTIES, CHUNK), jnp.int32),
          pltpu.VMEM((CHUNK, DIM_EMB), jnp.float32),
          pltpu.VMEM((CHUNK, DIM_EMB), jnp.float32),
          pltpu.VMEM((CHUNK, DIM_EMB), jnp.float32),
          pltpu.SemaphoreType.DMA,
          pltpu.SemaphoreType.DMA,
      ],
      compiler_params=pltpu.CompilerParams(use_tc_tiling_on_sc=False),
  )(xprep, table)


def _epi_body(pooled_ref, x_ref, mark_ref, emb_ref, pad_ref):
  s = jnp.sum(x_ref[...], axis=1, keepdims=True)
  pad = (s == 0)
  emb_ref[...] = jnp.where(pad, mark_ref[...],
                           pooled_ref[...] * (1.0 / N_PROPERTIES))
  pad_ref[...] = pad.astype(jnp.int32)


def _tc_epilogue(pooled, x2, mark):
  r_blk = 1024
  grid = (ROWS // r_blk,)
  return pl.pallas_call(
      _epi_body,
      grid=grid,
      in_specs=[
          pl.BlockSpec((r_blk, DIM_EMB), lambda i: (i, 0)),
          pl.BlockSpec((r_blk, N_PROPERTIES), lambda i: (i, 0)),
          pl.BlockSpec((1, DIM_EMB), lambda i: (0, 0)),
      ],
      out_specs=[
          pl.BlockSpec((r_blk, DIM_EMB), lambda i: (i, 0)),
          pl.BlockSpec((r_blk, 1), lambda i: (i, 0)),
      ],
      out_shape=[
          jax.ShapeDtypeStruct((ROWS, DIM_EMB), jnp.float32),
          jax.ShapeDtypeStruct((ROWS, 1), jnp.int32),
      ],
  )(pooled, x2, mark)


@jax.jit
def kernel(x, value_embedding, mark_absent, idx_offset):
  x2 = x.reshape(ROWS, N_PROPERTIES)
  xprep = x2.reshape(N_CHUNKS, CHUNK, N_PROPERTIES).transpose(0, 2, 1)
  pooled = _sc_gather_pool(xprep, value_embedding)
  emb, padi = _tc_epilogue(pooled, x2, mark_absent.reshape(1, DIM_EMB))
  bs, n_roles = x.shape[0], x.shape[1]
  return (emb.reshape(bs, n_roles, DIM_EMB),
          padi.reshape(bs, n_roles) != 0)
